# baseline (speedup 1.0000x reference)
import functools

import jax
import jax.numpy as jnp
from jax import lax
from jax.experimental import pallas as pl
from jax.experimental.pallas import tpu as pltpu
from jax.experimental.pallas import tpu_sc as plsc

N = 10000
NS = 5000
E = 320000
ES = 160000
M = 40000

NP = 10240
NOV = 10016
NSP = 5120
EP = 323584
ESP = 163840
MP = 40960
K = 128
NWORK = 32
TW = 144

_f32 = jnp.float32
_i32 = jnp.int32


def _mm(xm, w, b, block_rows):
    rows, din = xm.shape
    dout = w.shape[1]

    def body(x_ref, w_ref, b_ref, o_ref):
        o_ref[...] = jnp.dot(x_ref[...], w_ref[...],
                             preferred_element_type=_f32) + b_ref[...]

    return pl.pallas_call(
        body,
        grid=(rows // block_rows,),
        in_specs=[pl.BlockSpec((block_rows, din), lambda i: (i, 0)),
                  pl.BlockSpec((din, dout), lambda i: (0, 0)),
                  pl.BlockSpec((1, dout), lambda i: (0, 0))],
        out_specs=pl.BlockSpec((block_rows, dout), lambda i: (i, 0)),
        out_shape=jax.ShapeDtypeStruct((rows, dout), _f32),
    )(xm, w, b.reshape(1, dout))


def _sc_compute_edge(j, g0_v, g1_v, a_v, b1_v, w_v, out_v, mode):
    t0 = g0_v[j, pl.ds(128, 16)]
    off = 128 if mode == "graph" else 0
    t1 = g1_v[j, pl.ds(off, 16)]
    d = t1 - t0
    s = 1e-12 + d[0] * d[0] + d[1] * d[1] + d[2] * d[2]
    si = lax.bitcast_convert_type(s, _i32)
    yi = jnp.int32(0x5F3759DF) - lax.shift_right_logical(si, jnp.int32(1))
    y = lax.bitcast_convert_type(yi, _f32)
    half_s = 0.5 * s
    for _ in range(3):
        y = y * (1.5 - half_s * y * y)
    nrm = lax.broadcast(s * y, (16,))
    for c in range(8):
        sl = pl.ds(c * 16, 16)
        pre = g0_v[j, sl] + w_v[sl] * nrm
        if mode == "graph":
            pre = pre + g1_v[j, sl] + a_v[j, sl]
        else:
            pre = pre + b1_v[sl]
        out_v[j, sl] = jnp.maximum(pre, 0.0)
    ii = lax.iota(_i32, 16)
    lane3 = jnp.where(ii == 3, 1.0, 0.0).astype(_f32)
    out_v[j, pl.ds(128, 16)] = d + lane3


def _make_sc_edge(mode, ep, tab1_w, n_out, kc, ns=4):
    per_w = ep // NWORK
    n_chunks = per_w // kc
    assert n_chunks % ns == 0 and n_chunks >= 2 * ns, (ep, kc, ns)
    a_rows = kc if mode == "graph" else 8
    na = ns if mode == "graph" else 1
    rz = n_out // 16
    mesh = plsc.VectorSubcoreMesh(core_axis_name="c", subcore_axis_name="s")

    scratch = (
        [pltpu.VMEM((kc,), _i32) for _ in range(ns)]
        + [pltpu.VMEM((kc,), _i32) for _ in range(ns)]
        + [pltpu.VMEM((kc, TW), _f32) for _ in range(ns)]
        + [pltpu.VMEM((kc, tab1_w), _f32) for _ in range(ns)]
        + [pltpu.VMEM((a_rows, 128), _f32) for _ in range(na)]
        + [pltpu.VMEM((kc, TW), _f32) for _ in range(ns)]
        + [pltpu.VMEM((128,), _f32),
           pltpu.VMEM((128,), _f32),
           pltpu.VMEM_SHARED((n_out, TW), _f32)]
        + [pltpu.SemaphoreType.DMA] * (3 * ns + na)
    )

    @functools.partial(
        pl.kernel,
        out_type=jax.ShapeDtypeStruct((2, n_out, TW), _f32),
        mesh=mesh,
        compiler_params=pltpu.CompilerParams(use_tc_tiling_on_sc=False),
        scratch_types=scratch,
    )
    def k(t0_hbm, t1_hbm, a_hbm, w_hbm, b1_hbm, i0_hbm, i1_hbm,
          z_hbm, out_hbm, *scr):
        pos = [0]

        def take(n):
            r = scr[pos[0]:pos[0] + n]
            pos[0] += n
            return r

        i0_ = take(ns)
        i1_ = take(ns)
        g0_ = take(ns)
        g1_ = take(ns)
        av_ = take(na)
        out_ = take(ns)
        w_v, b1_v, accum = take(3)
        sg0_ = take(ns)
        sg1_ = take(ns)
        ss_ = take(ns)
        sa_ = take(na)

        sid = lax.axis_index("s")
        cid = lax.axis_index("c")
        wid = sid * 2 + cid

        pltpu.sync_copy(z_hbm, accum.at[pl.ds(sid * rz, rz)])
        pltpu.sync_copy(w_hbm, w_v)
        pltpu.sync_copy(b1_hbm, b1_v)
        plsc.subcore_barrier()

        base_w = wid * per_w

        def fire(ci, b):
            base = base_w + ci * kc
            pltpu.sync_copy(i0_hbm.at[pl.ds(base, kc)], i0_[b])
            pltpu.sync_copy(i1_hbm.at[pl.ds(base, kc)], i1_[b])
            pltpu.async_copy(t0_hbm.at[i0_[b]], g0_[b], sg0_[b])
            pltpu.async_copy(t1_hbm.at[i1_[b]], g1_[b], sg1_[b])
            if mode == "graph":
                pltpu.async_copy(a_hbm.at[pl.ds(base, kc)], av_[b], sa_[b])

        for i in range(ns - 1):
            fire(i, i)

        def group(ig, carry):
            for b in range(ns):
                ci = ig * ns + b
                b2 = (b + ns - 1) % ns

                @pl.when(ci > 0)
                def _():
                    pltpu.make_async_copy(
                        out_[b2], accum.at[i1_[b2]], ss_[b2]).wait()

                @pl.when(ci + ns - 1 < n_chunks)
                def _():
                    fire(ci + ns - 1, b2)

                pltpu.make_async_copy(t0_hbm.at[i0_[b]], g0_[b],
                                      sg0_[b]).wait()
                pltpu.make_async_copy(t1_hbm.at[i1_[b]], g1_[b],
                                      sg1_[b]).wait()
                if mode == "graph":
                    pltpu.make_async_copy(a_hbm.at[pl.ds(0, kc)], av_[b],
                                          sa_[b]).wait()

                @plsc.parallel_loop(0, kc, unroll=4)
                def _(j):
                    _sc_compute_edge(j, g0_[b], g1_[b], av_[b % na], b1_v,
                                     w_v, out_[b], mode)
                pltpu.async_copy(out_[b], accum.at[i1_[b]], ss_[b], add=True)
            return carry

        lax.fori_loop(0, n_chunks // ns, group, 0)
        bl = (n_chunks - 1) % ns
        pltpu.make_async_copy(out_[bl], accum.at[i1_[bl]], ss_[bl]).wait()
        plsc.subcore_barrier()
        sl = pl.ds(sid * rz, rz)
        pltpu.sync_copy(accum.at[sl], out_hbm.at[cid, sl])

    return k


def _combine(rm, rc, xin, w2m, b2m, w2c, b2c, wh1a, wh1b, bh1, wh2, bh2,
             block_rows):
    rows = rm.shape[1]

    def body(rm_ref, rc_ref, x_ref, w2m_ref, b2m_ref, w2c_ref, b2c_ref,
             a1_ref, a2_ref, bb1_ref, w2_ref, bb2_ref, x1_ref, h1_ref):
        R = rm_ref[0] + rm_ref[1]
        deg = R[:, 131:132]
        m_node = jnp.dot(R[:, :128], w2m_ref[...],
                         preferred_element_type=_f32) + deg * b2m_ref[...]
        x1_ref[...] = x_ref[...] + R[:, 128:131] / jnp.maximum(deg, 1.0)
        Rc = rc_ref[0] + rc_ref[1]
        cnt = Rc[:, 131:132]
        m_c = jnp.dot(Rc[:, :128], w2c_ref[...],
                      preferred_element_type=_f32) + cnt * b2c_ref[...]
        t = jnp.maximum(
            jnp.dot(m_node, a1_ref[...], preferred_element_type=_f32)
            + jnp.dot(m_c, a2_ref[...], preferred_element_type=_f32)
            + bb1_ref[...], 0.0)
        h1_ref[...] = jnp.dot(t, w2_ref[...],
                              preferred_element_type=_f32) + bb2_ref[...]

    full = lambda shape: pl.BlockSpec(shape, lambda i: tuple(0 for _ in shape))
    return pl.pallas_call(
        body,
        grid=(rows // block_rows,),
        in_specs=[pl.BlockSpec((2, block_rows, TW), lambda i: (0, i, 0)),
                  pl.BlockSpec((2, block_rows, TW), lambda i: (0, i, 0)),
                  pl.BlockSpec((block_rows, 3), lambda i: (i, 0)),
                  full((128, 128)), full((1, 128)),
                  full((128, 128)), full((1, 128)),
                  full((128, 128)), full((128, 128)), full((1, 128)),
                  full((128, 128)), full((1, 128))],
        out_specs=[pl.BlockSpec((block_rows, 3), lambda i: (i, 0)),
                   pl.BlockSpec((block_rows, 128), lambda i: (i, 0))],
        out_shape=[jax.ShapeDtypeStruct((rows, 3), _f32),
                   jax.ShapeDtypeStruct((rows, 128), _f32)],
    )(rm, rc, xin, w2m, b2m.reshape(1, 128), w2c, b2c.reshape(1, 128),
      wh1a, wh1b, bh1.reshape(1, 128), wh2, bh2.reshape(1, 128))


def _pad_rows(v, n, fill=0.0):
    pad = jnp.full((n - v.shape[0],) + v.shape[1:], fill, v.dtype)
    return jnp.concatenate([v, pad], axis=0)


def _pos16(xv, n):
    z = jnp.zeros((xv.shape[0], 13), _f32)
    return _pad_rows(jnp.concatenate([xv, z], axis=1), n)


def _pad_idx(iv, n, fill):
    return jnp.concatenate(
        [iv, jnp.full((n - iv.shape[0],), fill, _i32)], axis=0)


def kernel(x, h, a, x_star, h_star, a_star, params,
           edges, edges_star, cell_to_vertex_map, vertex_to_cell_map):
    p = params
    hp = _pad_rows(h, NP)
    hsp = _pad_rows(h_star, NSP)
    x16 = _pos16(x, NP)
    xs16 = _pos16(x_star, NSP)
    zN = jnp.zeros((NOV // 16, TW), _f32)
    zNS = jnp.zeros((NSP // 16, TW), _f32)
    zb = jnp.zeros((128,), _f32)
    z256 = jnp.zeros((256,), _f32)
    dummy_a = jnp.zeros((8, 128), _f32)

    W1e = p["phi_e"]["W1"]
    P01v = _mm(hp, jnp.concatenate([W1e[:128], W1e[128:256]], axis=1),
               z256, 512)
    ae = _pad_rows(a, EP)
    Ae = _mm(ae, W1e[257:273], p["phi_e"]["b1"], 1024)

    W1s = p["phi_star_e"]["W1"]
    P01s = _mm(hsp, jnp.concatenate([W1s[:128], W1s[128:256]], axis=1),
               z256, 512)
    asp = _pad_rows(a_star, ESP)
    As = _mm(asp, W1s[257:273], p["phi_star_e"]["b1"], 1024)

    W1vn = p["phi_v_n"]["W1"]
    Pvn = _mm(hsp, W1vn[:128], zb, 512)
    W1nv = p["phi_n_v"]["W1"]
    Pnv = _mm(hp, W1nv[:128], zb, 512)

    T0v = jnp.concatenate([P01v[:, :128], x16], axis=1)
    T1v = jnp.concatenate([P01v[:, 128:], x16], axis=1)
    T0s = jnp.concatenate([P01s[:, :128], xs16], axis=1)
    T1s = jnp.concatenate([P01s[:, 128:], xs16], axis=1)
    Tvn = jnp.concatenate([Pvn, xs16], axis=1)
    Tnv = jnp.concatenate([Pnv, x16], axis=1)

    e0 = _pad_idx(edges[:, 0], EP, N)
    e1 = _pad_idx(edges[:, 1], EP, N)
    s0 = _pad_idx(edges_star[:, 0], ESP, NS)
    s1 = _pad_idx(edges_star[:, 1], ESP, NS)
    v0 = _pad_idx(vertex_to_cell_map[:, 0], MP, N)
    v1 = _pad_idx(vertex_to_cell_map[:, 1], MP, 0)
    c0 = _pad_idx(cell_to_vertex_map[:, 0], MP, NS)
    c1 = _pad_idx(cell_to_vertex_map[:, 1], MP, 0)

    Rv = _make_sc_edge("graph", EP, TW, NOV, 16, 4)(
        T0v, T1v, Ae, W1e[256], zb, e0, e1, zN)
    Rs = _make_sc_edge("graph", ESP, TW, NSP, 32, 4)(
        T0s, T1s, As, W1s[256], zb, s0, s1, zNS)
    Rvc = _make_sc_edge("couple", MP, 16, NOV, 16, 4)(
        Tvn, x16, dummy_a, W1vn[128], p["phi_v_n"]["b1"], v1, v0, zN)
    Rcv = _make_sc_edge("couple", MP, 16, NSP, 32, 4)(
        Tnv, xs16, dummy_a, W1nv[128], p["phi_n_v"]["b1"], c1, c0, zNS)
    Rv = jnp.pad(Rv, ((0, 0), (0, NP - NOV), (0, 0)))
    Rvc = jnp.pad(Rvc, ((0, 0), (0, NP - NOV), (0, 0)))

    ph = p["phi_h"]
    x1, h1 = _combine(Rv, Rvc, _pad_rows(x, NP),
                      p["phi_e"]["W2"], p["phi_e"]["b2"],
                      p["phi_v_n"]["W2"], p["phi_v_n"]["b2"],
                      ph["W1"][:128], ph["W1"][128:], ph["b1"],
                      ph["W2"], ph["b2"], 512)
    phs = p["phi_h_star"]
    xs1, hs1 = _combine(Rs, Rcv, _pad_rows(x_star, NSP),
                        p["phi_star_e"]["W2"], p["phi_star_e"]["b2"],
                        p["phi_n_v"]["W2"], p["phi_n_v"]["b2"],
                        phs["W1"][:128], phs["W1"][128:], phs["b1"],
                        phs["W2"], phs["b2"], 512)

    return (x1[:N], h1[:N], xs1[:NS], hs1[:NS])


# speedup vs baseline: 1.3924x; 1.3924x over previous
import functools

import jax
import jax.numpy as jnp
from jax import lax
from jax.experimental import pallas as pl
from jax.experimental.pallas import tpu as pltpu
from jax.experimental.pallas import tpu_sc as plsc

N = 10000
NS = 5000
E = 320000
ES = 160000
M = 40000

NP = 10240
NOV = 10016
NSP = 5120
EP = 323584
ESP = 163840
MP = 40960
K = 128
NWORK = 32
TW = 144

_f32 = jnp.float32
_i32 = jnp.int32


def _mm(xm, w, b, block_rows):
    rows, din = xm.shape
    dout = w.shape[1]

    def body(x_ref, w_ref, b_ref, o_ref):
        o_ref[...] = jnp.dot(x_ref[...], w_ref[...],
                             preferred_element_type=_f32) + b_ref[...]

    return pl.pallas_call(
        body,
        grid=(rows // block_rows,),
        in_specs=[pl.BlockSpec((block_rows, din), lambda i: (i, 0)),
                  pl.BlockSpec((din, dout), lambda i: (0, 0)),
                  pl.BlockSpec((1, dout), lambda i: (0, 0))],
        out_specs=pl.BlockSpec((block_rows, dout), lambda i: (i, 0)),
        out_shape=jax.ShapeDtypeStruct((rows, dout), _f32),
    )(xm, w, b.reshape(1, dout))


def _sc_compute_edge(j, g0_v, g1_v, a_v, b1_v, w_v, out_v, mode):
    t0 = g0_v[j, pl.ds(128, 16)]
    off = 128 if mode == "graph" else 0
    t1 = g1_v[j, pl.ds(off, 16)]
    d = t1 - t0
    s = 1e-12 + d[0] * d[0] + d[1] * d[1] + d[2] * d[2]
    si = lax.bitcast_convert_type(s, _i32)
    yi = jnp.int32(0x5F3759DF) - lax.shift_right_logical(si, jnp.int32(1))
    y = lax.bitcast_convert_type(yi, _f32)
    half_s = 0.5 * s
    for _ in range(3):
        y = y * (1.5 - half_s * y * y)
    nrm = lax.broadcast(s * y, (16,))
    for c in range(8):
        sl = pl.ds(c * 16, 16)
        pre = g0_v[j, sl] + w_v[sl] * nrm
        if mode == "graph":
            pre = pre + g1_v[j, sl] + a_v[j, sl]
        else:
            pre = pre + b1_v[sl]
        out_v[j, sl] = jnp.maximum(pre, 0.0)
    ii = lax.iota(_i32, 16)
    lane3 = jnp.where(ii == 3, 1.0, 0.0).astype(_f32)
    out_v[j, pl.ds(128, 16)] = d + lane3


def _make_sc_edge(mode, ep, tab1_w, n_out, kc, ns=4):
    per_w = ep // NWORK
    n_chunks = per_w // kc
    assert ns == 2, ns
    assert n_chunks % (2 * ns) == 0 and n_chunks >= 2 * ns, (ep, kc, ns)
    a_rows = kc if mode == "graph" else 8
    na = ns if mode == "graph" else 1
    ni = 2 * ns
    rz = n_out // 16
    mesh = plsc.VectorSubcoreMesh(core_axis_name="c", subcore_axis_name="s")

    scratch = (
        [pltpu.VMEM((kc,), _i32) for _ in range(ni)]
        + [pltpu.VMEM((kc,), _i32) for _ in range(ni)]
        + [pltpu.VMEM((kc, TW), _f32) for _ in range(ns)]
        + [pltpu.VMEM((kc, tab1_w), _f32) for _ in range(ns)]
        + [pltpu.VMEM((a_rows, 128), _f32) for _ in range(na)]
        + [pltpu.VMEM((kc, TW), _f32) for _ in range(ns)]
        + [pltpu.VMEM((128,), _f32),
           pltpu.VMEM((128,), _f32),
           pltpu.VMEM_SHARED((n_out, TW), _f32)]
        + [pltpu.SemaphoreType.DMA] * (3 * ns + na + ni)
    )

    @functools.partial(
        pl.kernel,
        out_type=jax.ShapeDtypeStruct((2, n_out, TW), _f32),
        mesh=mesh,
        compiler_params=pltpu.CompilerParams(use_tc_tiling_on_sc=False),
        scratch_types=scratch,
    )
    def k(t0_hbm, t1_hbm, a_hbm, w_hbm, b1_hbm, i0_hbm, i1_hbm,
          z_hbm, out_hbm, *scr):
        pos = [0]

        def take(n):
            r = scr[pos[0]:pos[0] + n]
            pos[0] += n
            return r

        i0_ = take(ni)
        i1_ = take(ni)
        g0_ = take(ns)
        g1_ = take(ns)
        av_ = take(na)
        out_ = take(ns)
        w_v, b1_v, accum = take(3)
        sg0_ = take(ns)
        sg1_ = take(ns)
        ss_ = take(ns)
        sa_ = take(na)
        si_ = take(ni)

        sid = lax.axis_index("s")
        cid = lax.axis_index("c")
        wid = sid * 2 + cid

        pltpu.sync_copy(z_hbm, accum.at[pl.ds(sid * rz, rz)])
        pltpu.sync_copy(w_hbm, w_v)
        pltpu.sync_copy(b1_hbm, b1_v)
        plsc.subcore_barrier()

        base_w = wid * per_w

        def fire_idx(ci, bi):
            base = base_w + ci * kc
            pltpu.async_copy(i0_hbm.at[pl.ds(base, kc)], i0_[bi], si_[bi])
            pltpu.async_copy(i1_hbm.at[pl.ds(base, kc)], i1_[bi], si_[bi])

        def wait_idx(bi):
            pltpu.make_async_copy(i0_hbm.at[pl.ds(0, kc)], i0_[bi],
                                  si_[bi]).wait()
            pltpu.make_async_copy(i1_hbm.at[pl.ds(0, kc)], i1_[bi],
                                  si_[bi]).wait()

        def fire(ci, b, bi):
            base = base_w + ci * kc
            wait_idx(bi)
            pltpu.async_copy(t0_hbm.at[i0_[bi]], g0_[b], sg0_[b])
            pltpu.async_copy(t1_hbm.at[i1_[bi]], g1_[b], sg1_[b])
            if mode == "graph":
                pltpu.async_copy(a_hbm.at[pl.ds(base, kc)], av_[b], sa_[b])

        fire_idx(0, 0)
        fire_idx(1, 1)
        fire(0, 0, 0)

        def group(ig, carry):
            for b in range(ni):
                ci = ig * ni + b
                db = b % ns
                db2 = (b + 1) % ns

                @pl.when(ci > 0)
                def _():
                    pltpu.make_async_copy(
                        out_[db2], accum.at[i1_[(b + ni - 1) % ni]],
                        ss_[db2]).wait()

                @pl.when(ci + ns < n_chunks)
                def _():
                    fire_idx(ci + ns, (b + ns) % ni)

                @pl.when(ci + 1 < n_chunks)
                def _():
                    fire(ci + 1, db2, (b + 1) % ni)

                pltpu.make_async_copy(t0_hbm.at[i0_[b]], g0_[db],
                                      sg0_[db]).wait()
                pltpu.make_async_copy(t1_hbm.at[i1_[b]], g1_[db],
                                      sg1_[db]).wait()
                if mode == "graph":
                    pltpu.make_async_copy(a_hbm.at[pl.ds(0, kc)], av_[db],
                                          sa_[db]).wait()

                @plsc.parallel_loop(0, kc, unroll=4)
                def _(j):
                    _sc_compute_edge(j, g0_[db], g1_[db], av_[db % na], b1_v,
                                     w_v, out_[db], mode)
                pltpu.async_copy(out_[db], accum.at[i1_[b]], ss_[db],
                                 add=True)
            return carry

        lax.fori_loop(0, n_chunks // ni, group, 0)
        bl = (n_chunks - 1) % ns
        pltpu.make_async_copy(out_[bl], accum.at[i1_[(n_chunks - 1) % ni]],
                              ss_[bl]).wait()
        plsc.subcore_barrier()
        sl = pl.ds(sid * rz, rz)
        pltpu.sync_copy(accum.at[sl], out_hbm.at[cid, sl])

    return k


def _combine(rm, rc, xin, w2m, b2m, w2c, b2c, wh1a, wh1b, bh1, wh2, bh2,
             block_rows):
    rows = rm.shape[1]

    def body(rm_ref, rc_ref, x_ref, w2m_ref, b2m_ref, w2c_ref, b2c_ref,
             a1_ref, a2_ref, bb1_ref, w2_ref, bb2_ref, x1_ref, h1_ref):
        R = rm_ref[0] + rm_ref[1]
        deg = R[:, 131:132]
        m_node = jnp.dot(R[:, :128], w2m_ref[...],
                         preferred_element_type=_f32) + deg * b2m_ref[...]
        x1_ref[...] = x_ref[...] + R[:, 128:131] / jnp.maximum(deg, 1.0)
        Rc = rc_ref[0] + rc_ref[1]
        cnt = Rc[:, 131:132]
        m_c = jnp.dot(Rc[:, :128], w2c_ref[...],
                      preferred_element_type=_f32) + cnt * b2c_ref[...]
        t = jnp.maximum(
            jnp.dot(m_node, a1_ref[...], preferred_element_type=_f32)
            + jnp.dot(m_c, a2_ref[...], preferred_element_type=_f32)
            + bb1_ref[...], 0.0)
        h1_ref[...] = jnp.dot(t, w2_ref[...],
                              preferred_element_type=_f32) + bb2_ref[...]

    full = lambda shape: pl.BlockSpec(shape, lambda i: tuple(0 for _ in shape))
    return pl.pallas_call(
        body,
        grid=(rows // block_rows,),
        in_specs=[pl.BlockSpec((2, block_rows, TW), lambda i: (0, i, 0)),
                  pl.BlockSpec((2, block_rows, TW), lambda i: (0, i, 0)),
                  pl.BlockSpec((block_rows, 3), lambda i: (i, 0)),
                  full((128, 128)), full((1, 128)),
                  full((128, 128)), full((1, 128)),
                  full((128, 128)), full((128, 128)), full((1, 128)),
                  full((128, 128)), full((1, 128))],
        out_specs=[pl.BlockSpec((block_rows, 3), lambda i: (i, 0)),
                   pl.BlockSpec((block_rows, 128), lambda i: (i, 0))],
        out_shape=[jax.ShapeDtypeStruct((rows, 3), _f32),
                   jax.ShapeDtypeStruct((rows, 128), _f32)],
    )(rm, rc, xin, w2m, b2m.reshape(1, 128), w2c, b2c.reshape(1, 128),
      wh1a, wh1b, bh1.reshape(1, 128), wh2, bh2.reshape(1, 128))


def _pad_rows(v, n, fill=0.0):
    pad = jnp.full((n - v.shape[0],) + v.shape[1:], fill, v.dtype)
    return jnp.concatenate([v, pad], axis=0)


def _pos16(xv, n):
    z = jnp.zeros((xv.shape[0], 13), _f32)
    return _pad_rows(jnp.concatenate([xv, z], axis=1), n)


def _pad_idx(iv, n, fill):
    return jnp.concatenate(
        [iv, jnp.full((n - iv.shape[0],), fill, _i32)], axis=0)


def kernel(x, h, a, x_star, h_star, a_star, params,
           edges, edges_star, cell_to_vertex_map, vertex_to_cell_map):
    p = params
    hp = _pad_rows(h, NP)
    hsp = _pad_rows(h_star, NSP)
    x16 = _pos16(x, NP)
    xs16 = _pos16(x_star, NSP)
    zN = jnp.zeros((NOV // 16, TW), _f32)
    zNS = jnp.zeros((NSP // 16, TW), _f32)
    zb = jnp.zeros((128,), _f32)
    z256 = jnp.zeros((256,), _f32)
    dummy_a = jnp.zeros((8, 128), _f32)

    W1e = p["phi_e"]["W1"]
    P01v = _mm(hp, jnp.concatenate([W1e[:128], W1e[128:256]], axis=1),
               z256, 512)
    ae = _pad_rows(a, EP)
    Ae = _mm(ae, W1e[257:273], p["phi_e"]["b1"], 1024)

    W1s = p["phi_star_e"]["W1"]
    P01s = _mm(hsp, jnp.concatenate([W1s[:128], W1s[128:256]], axis=1),
               z256, 512)
    asp = _pad_rows(a_star, ESP)
    As = _mm(asp, W1s[257:273], p["phi_star_e"]["b1"], 1024)

    W1vn = p["phi_v_n"]["W1"]
    Pvn = _mm(hsp, W1vn[:128], zb, 512)
    W1nv = p["phi_n_v"]["W1"]
    Pnv = _mm(hp, W1nv[:128], zb, 512)

    T0v = jnp.concatenate([P01v[:, :128], x16], axis=1)
    T1v = jnp.concatenate([P01v[:, 128:], x16], axis=1)
    T0s = jnp.concatenate([P01s[:, :128], xs16], axis=1)
    T1s = jnp.concatenate([P01s[:, 128:], xs16], axis=1)
    Tvn = jnp.concatenate([Pvn, xs16], axis=1)
    Tnv = jnp.concatenate([Pnv, x16], axis=1)

    e0 = _pad_idx(edges[:, 0], EP, N)
    e1 = _pad_idx(edges[:, 1], EP, N)
    s0 = _pad_idx(edges_star[:, 0], ESP, NS)
    s1 = _pad_idx(edges_star[:, 1], ESP, NS)
    v0 = _pad_idx(vertex_to_cell_map[:, 0], MP, N)
    v1 = _pad_idx(vertex_to_cell_map[:, 1], MP, 0)
    c0 = _pad_idx(cell_to_vertex_map[:, 0], MP, NS)
    c1 = _pad_idx(cell_to_vertex_map[:, 1], MP, 0)

    Rv = _make_sc_edge("graph", EP, TW, NOV, 32, 2)(
        T0v, T1v, Ae, W1e[256], zb, e0, e1, zN)
    Rs = _make_sc_edge("graph", ESP, TW, NSP, 64, 2)(
        T0s, T1s, As, W1s[256], zb, s0, s1, zNS)
    Rvc = _make_sc_edge("couple", MP, 16, NOV, 32, 2)(
        Tvn, x16, dummy_a, W1vn[128], p["phi_v_n"]["b1"], v1, v0, zN)
    Rcv = _make_sc_edge("couple", MP, 16, NSP, 64, 2)(
        Tnv, xs16, dummy_a, W1nv[128], p["phi_n_v"]["b1"], c1, c0, zNS)
    Rv = jnp.pad(Rv, ((0, 0), (0, NP - NOV), (0, 0)))
    Rvc = jnp.pad(Rvc, ((0, 0), (0, NP - NOV), (0, 0)))

    ph = p["phi_h"]
    x1, h1 = _combine(Rv, Rvc, _pad_rows(x, NP),
                      p["phi_e"]["W2"], p["phi_e"]["b2"],
                      p["phi_v_n"]["W2"], p["phi_v_n"]["b2"],
                      ph["W1"][:128], ph["W1"][128:], ph["b1"],
                      ph["W2"], ph["b2"], 512)
    phs = p["phi_h_star"]
    xs1, hs1 = _combine(Rs, Rcv, _pad_rows(x_star, NSP),
                        p["phi_star_e"]["W2"], p["phi_star_e"]["b2"],
                        p["phi_n_v"]["W2"], p["phi_n_v"]["b2"],
                        phs["W1"][:128], phs["W1"][128:], phs["b1"],
                        phs["W2"], phs["b2"], 512)

    return (x1[:N], h1[:N], xs1[:NS], hs1[:NS])


# speedup vs baseline: 1.4940x; 1.0730x over previous
import functools

import jax
import jax.numpy as jnp
from jax import lax
from jax.experimental import pallas as pl
from jax.experimental.pallas import tpu as pltpu
from jax.experimental.pallas import tpu_sc as plsc

N = 10000
NS = 5000
E = 320000
ES = 160000
M = 40000

NP = 10240
NOV = 10016
NSP = 5120
EP = 323584
ESP = 163840
MP = 40960
K = 128
NWORK = 32
TW = 144

_f32 = jnp.float32
_i32 = jnp.int32


def _mm(xm, w, b, block_rows):
    rows, din = xm.shape
    dout = w.shape[1]

    def body(x_ref, w_ref, b_ref, o_ref):
        o_ref[...] = jnp.dot(x_ref[...], w_ref[...],
                             preferred_element_type=_f32) + b_ref[...]

    return pl.pallas_call(
        body,
        grid=(rows // block_rows,),
        in_specs=[pl.BlockSpec((block_rows, din), lambda i: (i, 0)),
                  pl.BlockSpec((din, dout), lambda i: (0, 0)),
                  pl.BlockSpec((1, dout), lambda i: (0, 0))],
        out_specs=pl.BlockSpec((block_rows, dout), lambda i: (i, 0)),
        out_shape=jax.ShapeDtypeStruct((rows, dout), _f32),
    )(xm, w, b.reshape(1, dout))


def _sc_compute_edge(j, g0_v, g1_v, a_v, b1_v, w_v, out_v, mode):
    t0 = g0_v[j, pl.ds(128, 16)]
    off = 128 if mode == "graph" else 0
    t1 = g1_v[j, pl.ds(off, 16)]
    d = t1 - t0
    s = 1e-12 + d[0] * d[0] + d[1] * d[1] + d[2] * d[2]
    si = lax.bitcast_convert_type(s, _i32)
    yi = jnp.int32(0x5F3759DF) - lax.shift_right_logical(si, jnp.int32(1))
    y = lax.bitcast_convert_type(yi, _f32)
    half_s = 0.5 * s
    for _ in range(3):
        y = y * (1.5 - half_s * y * y)
    nrm = lax.broadcast(s * y, (16,))
    for c in range(8):
        sl = pl.ds(c * 16, 16)
        pre = g0_v[j, sl] + w_v[sl] * nrm
        if mode == "graph":
            pre = pre + g1_v[j, sl] + a_v[j, sl]
        else:
            pre = pre + b1_v[sl]
        out_v[j, sl] = jnp.maximum(pre, 0.0)
    ii = lax.iota(_i32, 16)
    lane3 = jnp.where(ii == 3, 1.0, 0.0).astype(_f32)
    out_v[j, pl.ds(128, 16)] = d + lane3


def _sc_compute_edge_bf16(j, g0_v, g1_v, a_v, w_v, out_v):
    unpk = functools.partial(plsc.unpack, format=plsc.PackFormat.INTERLEAVED,
                             preferred_element_type=_f32)
    h0, l0 = unpk(g0_v[j, pl.ds(128, 32)])
    h1, l1 = unpk(g1_v[j, pl.ds(128, 32)])
    d = (h1 + l1) - (h0 + l0)
    s = 1e-12 + d[0] * d[0] + d[1] * d[1] + d[2] * d[2]
    si = lax.bitcast_convert_type(s, _i32)
    yi = jnp.int32(0x5F3759DF) - lax.shift_right_logical(si, jnp.int32(1))
    y = lax.bitcast_convert_type(yi, _f32)
    half_s = 0.5 * s
    for _ in range(3):
        y = y * (1.5 - half_s * y * y)
    nrm = lax.broadcast(s * y, (16,))
    for c in range(4):
        a0, b0 = unpk(g0_v[j, pl.ds(32 * c, 32)])
        a1, b1 = unpk(g1_v[j, pl.ds(32 * c, 32)])
        sla = pl.ds(32 * c, 16)
        slb = pl.ds(32 * c + 16, 16)
        out_v[j, sla] = jnp.maximum(
            a0 + a1 + a_v[j, sla] + w_v[sla] * nrm, 0.0)
        out_v[j, slb] = jnp.maximum(
            b0 + b1 + a_v[j, slb] + w_v[slb] * nrm, 0.0)
    ii = lax.iota(_i32, 16)
    lane3 = jnp.where(ii == 3, 1.0, 0.0).astype(_f32)
    out_v[j, pl.ds(128, 16)] = d + lane3


def _make_sc_edge(mode, ep, tab1_w, n_out, kc, ns=4):
    per_w = ep // NWORK
    n_chunks = per_w // kc
    assert ns == 2, ns
    assert n_chunks % (2 * ns) == 0 and n_chunks >= 2 * ns, (ep, kc, ns)
    a_rows = kc if mode == "graph" else 8
    na = ns if mode == "graph" else 1
    t0_w = 160 if mode == "graph" else TW
    t0_dt = jnp.bfloat16 if mode == "graph" else _f32
    t1_dt = jnp.bfloat16 if (mode == "graph" and tab1_w != 16) else _f32
    if mode == "graph":
        tab1_w = t0_w
    ni = 2 * ns
    rz = n_out // 16
    mesh = plsc.VectorSubcoreMesh(core_axis_name="c", subcore_axis_name="s")

    scratch = (
        [pltpu.VMEM((kc,), _i32) for _ in range(ni)]
        + [pltpu.VMEM((kc,), _i32) for _ in range(ni)]
        + [pltpu.VMEM((kc, t0_w), t0_dt) for _ in range(ns)]
        + [pltpu.VMEM((kc, tab1_w), t1_dt) for _ in range(ns)]
        + [pltpu.VMEM((a_rows, 128), _f32) for _ in range(na)]
        + [pltpu.VMEM((kc, TW), _f32) for _ in range(ns)]
        + [pltpu.VMEM((128,), _f32),
           pltpu.VMEM((128,), _f32),
           pltpu.VMEM_SHARED((n_out, TW), _f32)]
        + [pltpu.SemaphoreType.DMA] * (3 * ns + na + ni)
    )

    @functools.partial(
        pl.kernel,
        out_type=jax.ShapeDtypeStruct((2, n_out, TW), _f32),
        mesh=mesh,
        compiler_params=pltpu.CompilerParams(use_tc_tiling_on_sc=False, needs_layout_passes=False),
        scratch_types=scratch,
    )
    def k(t0_hbm, t1_hbm, a_hbm, w_hbm, b1_hbm, i0_hbm, i1_hbm,
          z_hbm, out_hbm, *scr):
        pos = [0]

        def take(n):
            r = scr[pos[0]:pos[0] + n]
            pos[0] += n
            return r

        i0_ = take(ni)
        i1_ = take(ni)
        g0_ = take(ns)
        g1_ = take(ns)
        av_ = take(na)
        out_ = take(ns)
        w_v, b1_v, accum = take(3)
        sg0_ = take(ns)
        sg1_ = take(ns)
        ss_ = take(ns)
        sa_ = take(na)
        si_ = take(ni)

        sid = lax.axis_index("s")
        cid = lax.axis_index("c")
        wid = sid * 2 + cid

        pltpu.sync_copy(z_hbm, accum.at[pl.ds(sid * rz, rz)])
        pltpu.sync_copy(w_hbm, w_v)
        pltpu.sync_copy(b1_hbm, b1_v)
        plsc.subcore_barrier()

        base_w = wid * per_w

        def fire_idx(ci, bi):
            base = base_w + ci * kc
            pltpu.async_copy(i0_hbm.at[pl.ds(base, kc)], i0_[bi], si_[bi])
            pltpu.async_copy(i1_hbm.at[pl.ds(base, kc)], i1_[bi], si_[bi])

        def wait_idx(bi):
            pltpu.make_async_copy(i0_hbm.at[pl.ds(0, kc)], i0_[bi],
                                  si_[bi]).wait()
            pltpu.make_async_copy(i1_hbm.at[pl.ds(0, kc)], i1_[bi],
                                  si_[bi]).wait()

        def fire(ci, b, bi):
            base = base_w + ci * kc
            wait_idx(bi)
            pltpu.async_copy(t0_hbm.at[i0_[bi]], g0_[b], sg0_[b])
            pltpu.async_copy(t1_hbm.at[i1_[bi]], g1_[b], sg1_[b])
            if mode == "graph":
                pltpu.async_copy(a_hbm.at[pl.ds(base, kc)], av_[b], sa_[b])

        fire_idx(0, 0)
        fire_idx(1, 1)
        fire(0, 0, 0)

        def group(ig, carry):
            for b in range(ni):
                ci = ig * ni + b
                db = b % ns
                db2 = (b + 1) % ns

                @pl.when(ci > 0)
                def _():
                    pltpu.make_async_copy(
                        out_[db2], accum.at[i1_[(b + ni - 1) % ni]],
                        ss_[db2]).wait()

                @pl.when(ci + ns < n_chunks)
                def _():
                    fire_idx(ci + ns, (b + ns) % ni)

                @pl.when(ci + 1 < n_chunks)
                def _():
                    fire(ci + 1, db2, (b + 1) % ni)

                pltpu.make_async_copy(t0_hbm.at[i0_[b]], g0_[db],
                                      sg0_[db]).wait()
                pltpu.make_async_copy(t1_hbm.at[i1_[b]], g1_[db],
                                      sg1_[db]).wait()
                if mode == "graph":
                    pltpu.make_async_copy(a_hbm.at[pl.ds(0, kc)], av_[db],
                                          sa_[db]).wait()

                @plsc.parallel_loop(0, kc, unroll=4)
                def _(j):
                    if mode == "graph":
                        _sc_compute_edge_bf16(j, g0_[db], g1_[db],
                                              av_[db % na], w_v, out_[db])
                    else:
                        _sc_compute_edge(j, g0_[db], g1_[db], av_[db % na],
                                         b1_v, w_v, out_[db], mode)
                pltpu.async_copy(out_[db], accum.at[i1_[b]], ss_[db],
                                 add=True)
            return carry

        lax.fori_loop(0, n_chunks // ni, group, 0)
        bl = (n_chunks - 1) % ns
        pltpu.make_async_copy(out_[bl], accum.at[i1_[(n_chunks - 1) % ni]],
                              ss_[bl]).wait()
        plsc.subcore_barrier()
        sl = pl.ds(sid * rz, rz)
        pltpu.sync_copy(accum.at[sl], out_hbm.at[cid, sl])

    return k


def _combine(rm, rc, xin, w2m, b2m, w2c, b2c, wh1a, wh1b, bh1, wh2, bh2,
             block_rows):
    rows = rm.shape[1]

    def body(rm_ref, rc_ref, x_ref, w2m_ref, b2m_ref, w2c_ref, b2c_ref,
             a1_ref, a2_ref, bb1_ref, w2_ref, bb2_ref, x1_ref, h1_ref):
        R = rm_ref[0] + rm_ref[1]
        deg = R[:, 131:132]
        m_node = jnp.dot(R[:, :128], w2m_ref[...],
                         preferred_element_type=_f32) + deg * b2m_ref[...]
        x1_ref[...] = x_ref[...] + R[:, 128:131] / jnp.maximum(deg, 1.0)
        Rc = rc_ref[0] + rc_ref[1]
        cnt = Rc[:, 131:132]
        m_c = jnp.dot(Rc[:, :128], w2c_ref[...],
                      preferred_element_type=_f32) + cnt * b2c_ref[...]
        t = jnp.maximum(
            jnp.dot(m_node, a1_ref[...], preferred_element_type=_f32)
            + jnp.dot(m_c, a2_ref[...], preferred_element_type=_f32)
            + bb1_ref[...], 0.0)
        h1_ref[...] = jnp.dot(t, w2_ref[...],
                              preferred_element_type=_f32) + bb2_ref[...]

    full = lambda shape: pl.BlockSpec(shape, lambda i: tuple(0 for _ in shape))
    return pl.pallas_call(
        body,
        grid=(rows // block_rows,),
        in_specs=[pl.BlockSpec((2, block_rows, TW), lambda i: (0, i, 0)),
                  pl.BlockSpec((2, block_rows, TW), lambda i: (0, i, 0)),
                  pl.BlockSpec((block_rows, 3), lambda i: (i, 0)),
                  full((128, 128)), full((1, 128)),
                  full((128, 128)), full((1, 128)),
                  full((128, 128)), full((128, 128)), full((1, 128)),
                  full((128, 128)), full((1, 128))],
        out_specs=[pl.BlockSpec((block_rows, 3), lambda i: (i, 0)),
                   pl.BlockSpec((block_rows, 128), lambda i: (i, 0))],
        out_shape=[jax.ShapeDtypeStruct((rows, 3), _f32),
                   jax.ShapeDtypeStruct((rows, 128), _f32)],
    )(rm, rc, xin, w2m, b2m.reshape(1, 128), w2c, b2c.reshape(1, 128),
      wh1a, wh1b, bh1.reshape(1, 128), wh2, bh2.reshape(1, 128))


def _pad_rows(v, n, fill=0.0):
    pad = jnp.full((n - v.shape[0],) + v.shape[1:], fill, v.dtype)
    return jnp.concatenate([v, pad], axis=0)


def _pos16(xv, n):
    z = jnp.zeros((xv.shape[0], 13), _f32)
    return _pad_rows(jnp.concatenate([xv, z], axis=1), n)


def _pad_idx(iv, n, fill):
    return jnp.concatenate(
        [iv, jnp.full((n - iv.shape[0],), fill, _i32)], axis=0)


def kernel(x, h, a, x_star, h_star, a_star, params,
           edges, edges_star, cell_to_vertex_map, vertex_to_cell_map):
    p = params
    hp = _pad_rows(h, NP)
    hsp = _pad_rows(h_star, NSP)
    x16 = _pos16(x, NP)
    xs16 = _pos16(x_star, NSP)
    zN = jnp.zeros((NOV // 16, TW), _f32)
    zNS = jnp.zeros((NSP // 16, TW), _f32)
    zb = jnp.zeros((128,), _f32)
    z256 = jnp.zeros((256,), _f32)
    dummy_a = jnp.zeros((8, 128), _f32)

    W1e = p["phi_e"]["W1"]
    P01v = _mm(hp, jnp.concatenate([W1e[:128], W1e[128:256]], axis=1),
               z256, 512)
    ae = _pad_rows(a, EP)
    Ae = _mm(ae, W1e[257:273], p["phi_e"]["b1"], 1024)

    W1s = p["phi_star_e"]["W1"]
    P01s = _mm(hsp, jnp.concatenate([W1s[:128], W1s[128:256]], axis=1),
               z256, 512)
    asp = _pad_rows(a_star, ESP)
    As = _mm(asp, W1s[257:273], p["phi_star_e"]["b1"], 1024)

    W1vn = p["phi_v_n"]["W1"]
    Pvn = _mm(hsp, W1vn[:128], zb, 512)
    W1nv = p["phi_n_v"]["W1"]
    Pnv = _mm(hp, W1nv[:128], zb, 512)

    def _bf16_table(proj, x16v):
        rows = proj.shape[0]
        pint = proj.reshape(rows, 4, 2, 16).transpose(0, 1, 3, 2)
        pint = pint.reshape(rows, 128).astype(jnp.bfloat16)
        xhi = x16v.astype(jnp.bfloat16)
        xlo = (x16v - xhi.astype(_f32)).astype(jnp.bfloat16)
        tail = jnp.stack([xhi, xlo], axis=-1).reshape(rows, 32)
        return jnp.concatenate([pint, tail], axis=1)

    T0v = _bf16_table(P01v[:, :128], x16)
    T1v = _bf16_table(P01v[:, 128:], x16)
    T0s = _bf16_table(P01s[:, :128], xs16)
    T1s = _bf16_table(P01s[:, 128:], xs16)
    Tvn = jnp.concatenate([Pvn, xs16], axis=1)
    Tnv = jnp.concatenate([Pnv, x16], axis=1)

    e0 = _pad_idx(edges[:, 0], EP, N)
    e1 = _pad_idx(edges[:, 1], EP, N)
    s0 = _pad_idx(edges_star[:, 0], ESP, NS)
    s1 = _pad_idx(edges_star[:, 1], ESP, NS)
    v0 = _pad_idx(vertex_to_cell_map[:, 0], MP, N)
    v1 = _pad_idx(vertex_to_cell_map[:, 1], MP, 0)
    c0 = _pad_idx(cell_to_vertex_map[:, 0], MP, NS)
    c1 = _pad_idx(cell_to_vertex_map[:, 1], MP, 0)

    Rv = _make_sc_edge("graph", EP, TW, NOV, 32, 2)(
        T0v, T1v, Ae, W1e[256], zb, e0, e1, zN)
    Rs = _make_sc_edge("graph", ESP, TW, NSP, 64, 2)(
        T0s, T1s, As, W1s[256], zb, s0, s1, zNS)
    Rvc = _make_sc_edge("couple", MP, 16, NOV, 32, 2)(
        Tvn, x16, dummy_a, W1vn[128], p["phi_v_n"]["b1"], v1, v0, zN)
    Rcv = _make_sc_edge("couple", MP, 16, NSP, 64, 2)(
        Tnv, xs16, dummy_a, W1nv[128], p["phi_n_v"]["b1"], c1, c0, zNS)
    Rv = jnp.pad(Rv, ((0, 0), (0, NP - NOV), (0, 0)))
    Rvc = jnp.pad(Rvc, ((0, 0), (0, NP - NOV), (0, 0)))

    ph = p["phi_h"]
    x1, h1 = _combine(Rv, Rvc, _pad_rows(x, NP),
                      p["phi_e"]["W2"], p["phi_e"]["b2"],
                      p["phi_v_n"]["W2"], p["phi_v_n"]["b2"],
                      ph["W1"][:128], ph["W1"][128:], ph["b1"],
                      ph["W2"], ph["b2"], 512)
    phs = p["phi_h_star"]
    xs1, hs1 = _combine(Rs, Rcv, _pad_rows(x_star, NSP),
                        p["phi_star_e"]["W2"], p["phi_star_e"]["b2"],
                        p["phi_n_v"]["W2"], p["phi_n_v"]["b2"],
                        phs["W1"][:128], phs["W1"][128:], phs["b1"],
                        phs["W2"], phs["b2"], 512)

    return (x1[:N], h1[:N], xs1[:NS], hs1[:NS])
